# Initial kernel scaffold; baseline (speedup 1.0000x reference)
import jax
import jax.numpy as jnp
from jax.experimental import pallas as pl


def kernel(user_x, prod_x, edge_index, edge_label_index, W_user, b_user, W_prod, b_prod, W_l1_buy, b_l1_buy, W_r1_buy, W_l1_rev, b_l1_rev, W_r1_rev, W_l2_buy, b_l2_buy, W_r2_buy, W_l2_rev, b_l2_rev, W_r2_rev, W_l3_buy, b_l3_buy, W_r3_buy, W_l3_rev, b_l3_rev, W_r3_rev):
    raise NotImplementedError("write your pallas kernel here")


# speedup vs baseline: 6.3854x; 6.3854x over previous
import functools

import jax
import jax.numpy as jnp
from jax import lax
from jax.experimental import pallas as pl
from jax.experimental.pallas import tpu as pltpu
from jax.experimental.pallas import tpu_sc as plsc

N = 50000
D = 128
E = 625000
L = 100000

N_PAD = 50176
ROWS_PER_TILE = N_PAD // 16
CHUNK = 128
G = 2
CHUNKS_PER_TILE = 312
GROUPS = CHUNKS_PER_TILE // G
HALF_GROUPS = GROUPS // 2
E_PAD = 16 * CHUNKS_PER_TILE * CHUNK
L_PAD = 100352
LCHUNK = 112
LCHUNKS = 28

_BLK = 512
_GRID = N_PAD // _BLK

_mesh = plsc.VectorSubcoreMesh(core_axis_name="c", subcore_axis_name="s")
_f32 = jnp.float32
_sc_params = pltpu.CompilerParams(use_tc_tiling_on_sc=False,
                                  needs_layout_passes=False)


def _init_body(ux, wu, bu, px, wp, bp, ou, op):
    ou[...] = jnp.dot(ux[...], wu[...], preferred_element_type=_f32,
                      precision=lax.Precision.HIGHEST) + bu[...]
    op[...] = jnp.dot(px[...], wp[...], preferred_element_type=_f32,
                      precision=lax.Precision.HIGHEST) + bp[...]


def _tc_init(ux, wu, bu, px, wp, bp):
    spec_x = pl.BlockSpec((_BLK, D), lambda i: (i, 0))
    spec_w = pl.BlockSpec((D, D), lambda i: (0, 0))
    spec_b = pl.BlockSpec((1, D), lambda i: (0, 0))
    return pl.pallas_call(
        _init_body,
        grid=(_GRID,),
        in_specs=[spec_x, spec_w, spec_b, spec_x, spec_w, spec_b],
        out_specs=[spec_x, spec_x],
        out_shape=[jax.ShapeDtypeStruct((N_PAD, D), _f32)] * 2,
    )(ux, wu, bu.reshape(1, D), px, wp, bp.reshape(1, D))


def _update_body(relu, sp, su, cnt, u, p, wlb, blb, wrb, wlr, blr, wrr,
                 op, ou):
    mean_p = sp[...]
    mean_u = su[...]
    c = cnt[...]
    rp = 1.0 / jnp.maximum(c[0, :, 0:1], 1.0)
    ru = 1.0 / jnp.maximum(c[1, :, 0:1], 1.0)
    pn = (jnp.dot(mean_p * rp, wlb[...], preferred_element_type=_f32,
                  precision=lax.Precision.HIGHEST) + blb[...]
          + jnp.dot(p[...], wrb[...], preferred_element_type=_f32,
                    precision=lax.Precision.HIGHEST))
    un = (jnp.dot(mean_u * ru, wlr[...], preferred_element_type=_f32,
                  precision=lax.Precision.HIGHEST) + blr[...]
          + jnp.dot(u[...], wrr[...], preferred_element_type=_f32,
                    precision=lax.Precision.HIGHEST))
    if relu:
        pn = jnp.maximum(pn, 0.0)
        un = jnp.maximum(un, 0.0)
    op[...] = pn
    ou[...] = un


def _tc_update(relu, sp, su, cnt, u, p, wlb, blb, wrb, wlr, blr, wrr):
    spec_c = pl.BlockSpec((2, _BLK, 16), lambda i: (0, i, 0))
    spec_x = pl.BlockSpec((_BLK, D), lambda i: (i, 0))
    spec_w = pl.BlockSpec((D, D), lambda i: (0, 0))
    spec_b = pl.BlockSpec((1, D), lambda i: (0, 0))
    return pl.pallas_call(
        functools.partial(_update_body, relu),
        grid=(_GRID,),
        in_specs=[spec_x, spec_x, spec_c, spec_x, spec_x,
                  spec_w, spec_b, spec_w, spec_w, spec_b, spec_w],
        out_specs=[spec_x, spec_x],
        out_shape=[jax.ShapeDtypeStruct((N_PAD, D), _f32)] * 2,
    )(sp, su, cnt, u, p, wlb, blb.reshape(1, D), wrb,
      wlr, blr.reshape(1, D), wrr)


@functools.partial(
    pl.kernel,
    out_type=jax.ShapeDtypeStruct((2, N_PAD, 16), _f32),
    mesh=_mesh,
    compiler_params=_sc_params,
    scratch_types=[
        pltpu.VMEM_SHARED((N_PAD, 16), _f32),
        pltpu.VMEM((CHUNK,), jnp.int32),
        pltpu.VMEM((CHUNK, 16), _f32),
    ],
)
def _sc_counts(esrc, edst, zeros16, out, acc, dbuf, ones):
    c = lax.axis_index("c")
    s = lax.axis_index("s")

    @pl.loop(0, CHUNK)
    def _(i):
        ones[i, :] = jnp.ones((16,), _f32)

    pltpu.sync_copy(zeros16.at[pl.ds(s * ROWS_PER_TILE, ROWS_PER_TILE)],
                    acc.at[pl.ds(s * ROWS_PER_TILE, ROWS_PER_TILE)])
    plsc.subcore_barrier()

    @pl.loop(0, CHUNKS_PER_TILE)
    def _(ci):
        base = (s * CHUNKS_PER_TILE + ci) * CHUNK

        @pl.when(c == 0)
        def _():
            pltpu.sync_copy(edst.at[pl.ds(base, CHUNK)], dbuf)

        @pl.when(c == 1)
        def _():
            pltpu.sync_copy(esrc.at[pl.ds(base, CHUNK)], dbuf)

        pltpu.sync_copy(ones, acc.at[dbuf], add=True)

    plsc.subcore_barrier()
    pltpu.sync_copy(acc.at[pl.ds(s * ROWS_PER_TILE, ROWS_PER_TILE)],
                    out.at[c, pl.ds(s * ROWS_PER_TILE, ROWS_PER_TILE)])


@functools.partial(
    pl.kernel,
    out_type=[jax.ShapeDtypeStruct((N_PAD, D), _f32)] * 2,
    mesh=_mesh,
    compiler_params=_sc_params,
    scratch_types=[
        pltpu.VMEM_SHARED((N_PAD, 32), _f32),
        pltpu.VMEM((2, G, CHUNK), jnp.int32),
        pltpu.VMEM((2, G, CHUNK), jnp.int32),
        pltpu.VMEM((2, G, CHUNK), jnp.int32),
        pltpu.VMEM((2, G, CHUNK, 32), _f32),
        pltpu.SemaphoreType.DMA((2,)),
        pltpu.SemaphoreType.DMA((2,)),
        pltpu.SemaphoreType.DMA((2,)),
    ],
)
def _sc_segsum(u4, p4, esrc, edst, zeros32, sp_out, su_out,
               acc, sbuf, gbuf, dbuf, rbuf, sem_i, sem_g, sem_a):
    c = lax.axis_index("c")
    s = lax.axis_index("s")

    for d in range(2):
        tab = u4 if d == 0 else p4
        src_h = esrc if d == 0 else edst
        dst_h = edst if d == 0 else esrc
        out_h = sp_out if d == 0 else su_out
        for r in range(2):
            q = r * 2 + c
            pltpu.sync_copy(zeros32.at[pl.ds(s * ROWS_PER_TILE, ROWS_PER_TILE)],
                            acc.at[pl.ds(s * ROWS_PER_TILE, ROWS_PER_TILE)])
            plsc.subcore_barrier()
            tile_base = s * CHUNKS_PER_TILE * CHUNK

            def idx_start(gi, ph, j):
                b = tile_base + (gi * G + j) * CHUNK
                pltpu.async_copy(src_h.at[pl.ds(b, CHUNK)], sbuf.at[ph, j],
                                 sem_i.at[ph])
                pltpu.async_copy(dst_h.at[pl.ds(b, CHUNK)], dbuf.at[ph, j],
                                 sem_i.at[ph])

            def idx_wait(ph, j):
                pltpu.make_async_copy(src_h.at[pl.ds(0, CHUNK)],
                                      sbuf.at[ph, j], sem_i.at[ph]).wait()
                pltpu.make_async_copy(dst_h.at[pl.ds(0, CHUNK)],
                                      dbuf.at[ph, j], sem_i.at[ph]).wait()

            def add_wait(ph, j):
                pltpu.make_async_copy(rbuf.at[ph, j], acc.at[dbuf.at[ph, j]],
                                      sem_a.at[ph]).wait()

            def group_body(gi, ph):
                for j in range(G):
                    idx_wait(ph, j)
                for j in range(G):
                    @pl.loop(0, CHUNK, step=16)
                    def _(i):
                        gbuf[ph, j, pl.ds(i, 16)] = (
                            sbuf[ph, j, pl.ds(i, 16)] * 4 + q)
                @pl.when(gi >= 2)
                def _():
                    for j in range(G):
                        add_wait(ph, j)
                for j in range(G):
                    pltpu.async_copy(tab.at[gbuf.at[ph, j]], rbuf.at[ph, j],
                                     sem_g.at[ph])
                @pl.when(gi + 1 < GROUPS)
                def _():
                    for j in range(G):
                        idx_start(gi + 1, 1 - ph, j)
                for j in range(G):
                    pltpu.make_async_copy(tab.at[gbuf.at[ph, j]],
                                          rbuf.at[ph, j], sem_g.at[ph]).wait()
                for j in range(G):
                    pltpu.async_copy(rbuf.at[ph, j], acc.at[dbuf.at[ph, j]],
                                     sem_a.at[ph], add=True)

            for j in range(G):
                idx_start(0, 0, j)

            @pl.loop(0, HALF_GROUPS)
            def _(h):
                group_body(2 * h, 0)
                group_body(2 * h + 1, 1)

            for ph in range(2):
                for j in range(G):
                    add_wait(ph, j)
            plsc.subcore_barrier()
            pltpu.sync_copy(acc.at[pl.ds(s * ROWS_PER_TILE, ROWS_PER_TILE)],
                            out_h.at[pl.ds(s * ROWS_PER_TILE, ROWS_PER_TILE),
                                     pl.ds(q * 32, 32)])


@functools.partial(
    pl.kernel,
    out_type=jax.ShapeDtypeStruct((L_PAD,), _f32),
    mesh=_mesh,
    compiler_params=_sc_params,
    scratch_types=[
        pltpu.VMEM((LCHUNK,), jnp.int32),
        pltpu.VMEM((LCHUNK,), jnp.int32),
        pltpu.VMEM((LCHUNK, D), _f32),
        pltpu.VMEM((LCHUNK, D), _f32),
        pltpu.VMEM((ROWS_PER_TILE,), _f32),
    ],
)
def _sc_classifier(u, p, eli0, eli1, out, i0, i1, ubuf, pbuf, obuf):
    c = lax.axis_index("c")
    s = lax.axis_index("s")
    w = c * 16 + s
    lanes = lax.broadcasted_iota(jnp.int32, (16,), 0)

    @pl.loop(0, LCHUNKS)
    def _(k):
        base = w * ROWS_PER_TILE + k * LCHUNK
        pltpu.sync_copy(eli0.at[pl.ds(base, LCHUNK)], i0)
        pltpu.sync_copy(eli1.at[pl.ds(base, LCHUNK)], i1)
        pltpu.sync_copy(u.at[i0], ubuf)
        pltpu.sync_copy(p.at[i1], pbuf)

        @pl.loop(0, LCHUNK, step=16)
        def _(g):
            row_idx = g + lanes
            acc0 = jnp.zeros((16,), _f32)

            def body(j, acc):
                col_idx = jnp.full((16,), 0, jnp.int32) + j
                uc = plsc.load_gather(ubuf, [row_idx, col_idx])
                pc = plsc.load_gather(pbuf, [row_idx, col_idx])
                return acc + uc * pc

            accv = lax.fori_loop(0, D, body, acc0)
            obuf[pl.ds(k * LCHUNK + g, 16)] = accv

    pltpu.sync_copy(obuf, out.at[pl.ds(w * ROWS_PER_TILE, ROWS_PER_TILE)])


def kernel(user_x, prod_x, edge_index, edge_label_index,
           W_user, b_user, W_prod, b_prod,
           W_l1_buy, b_l1_buy, W_r1_buy, W_l1_rev, b_l1_rev, W_r1_rev,
           W_l2_buy, b_l2_buy, W_r2_buy, W_l2_rev, b_l2_rev, W_r2_rev,
           W_l3_buy, b_l3_buy, W_r3_buy, W_l3_rev, b_l3_rev, W_r3_rev):
    ux = jnp.pad(user_x, ((0, N_PAD - N), (0, 0)))
    px = jnp.pad(prod_x, ((0, N_PAD - N), (0, 0)))

    npad = E_PAD - E
    fill = jnp.arange(npad, dtype=jnp.int32)
    esrc = jnp.concatenate([edge_index[0], N + fill % (N_PAD - N)])
    edst = jnp.concatenate([edge_index[1], N + (fill * 7 + 3) % (N_PAD - N)])

    zeros32 = jnp.zeros((N_PAD, 32), _f32)
    zeros16 = jnp.zeros((N_PAD, 16), _f32)

    cnt = _sc_counts(esrc, edst, zeros16)

    u, p = _tc_init(ux, W_user, b_user, px, W_prod, b_prod)

    layers = [
        (W_l1_buy, b_l1_buy, W_r1_buy, W_l1_rev, b_l1_rev, W_r1_rev),
        (W_l2_buy, b_l2_buy, W_r2_buy, W_l2_rev, b_l2_rev, W_r2_rev),
        (W_l3_buy, b_l3_buy, W_r3_buy, W_l3_rev, b_l3_rev, W_r3_rev),
    ]
    for i, (wlb, blb, wrb, wlr, blr, wrr) in enumerate(layers):
        sp, su = _sc_segsum(u.reshape(4 * N_PAD, 32), p.reshape(4 * N_PAD, 32),
                            esrc, edst, zeros32)
        p, u = _tc_update(i < 2, sp, su, cnt, u, p,
                          wlb, blb, wrb, wlr, blr, wrr)

    lpad = L_PAD - L
    lfill = jnp.arange(lpad, dtype=jnp.int32) % N
    eli0 = jnp.concatenate([edge_label_index[0], lfill])
    eli1 = jnp.concatenate([edge_label_index[1], lfill])
    pred = _sc_classifier(u, p, eli0, eli1)
    return pred[:L]


# speedup vs baseline: 7.1937x; 1.1266x over previous
import functools

import jax
import jax.numpy as jnp
from jax import lax
from jax.experimental import pallas as pl
from jax.experimental.pallas import tpu as pltpu
from jax.experimental.pallas import tpu_sc as plsc

N = 50000
D = 128
E = 625000
L = 100000

N_PAD = 50176
ROWS_PER_TILE = N_PAD // 16
CHUNK = 256
G = 1
CHUNKS_PER_TILE = 156
GROUPS = CHUNKS_PER_TILE // G
HALF_GROUPS = GROUPS // 2
E_PAD = 16 * CHUNKS_PER_TILE * CHUNK
L_PAD = 100352
LCHUNK = 112
LCHUNKS = 28

_BLK = 512
_GRID = N_PAD // _BLK

_mesh = plsc.VectorSubcoreMesh(core_axis_name="c", subcore_axis_name="s")
_f32 = jnp.float32
_sc_params = pltpu.CompilerParams(use_tc_tiling_on_sc=False,
                                  needs_layout_passes=False)


def _init_body(ux, wu, bu, px, wp, bp, ou, op):
    ou[...] = jnp.dot(ux[...], wu[...], preferred_element_type=_f32) + bu[...]
    op[...] = jnp.dot(px[...], wp[...], preferred_element_type=_f32) + bp[...]


def _tc_init(ux, wu, bu, px, wp, bp):
    spec_x = pl.BlockSpec((_BLK, D), lambda i: (i, 0))
    spec_w = pl.BlockSpec((D, D), lambda i: (0, 0))
    spec_b = pl.BlockSpec((1, D), lambda i: (0, 0))
    return pl.pallas_call(
        _init_body,
        grid=(_GRID,),
        in_specs=[spec_x, spec_w, spec_b, spec_x, spec_w, spec_b],
        out_specs=[spec_x, spec_x],
        out_shape=[jax.ShapeDtypeStruct((N_PAD, D), _f32)] * 2,
    )(ux, wu, bu.reshape(1, D), px, wp, bp.reshape(1, D))


def _update_body(relu, sp, su, cnt, u, p, wlb, blb, wrb, wlr, blr, wrr,
                 op, ou):
    mean_p = sp[...]
    mean_u = su[...]
    c = cnt[...]
    rp = 1.0 / jnp.maximum(c[0, :, 0:1], 1.0)
    ru = 1.0 / jnp.maximum(c[1, :, 0:1], 1.0)
    pn = (jnp.dot(mean_p * rp, wlb[...], preferred_element_type=_f32) + blb[...]
          + jnp.dot(p[...], wrb[...], preferred_element_type=_f32))
    un = (jnp.dot(mean_u * ru, wlr[...], preferred_element_type=_f32) + blr[...]
          + jnp.dot(u[...], wrr[...], preferred_element_type=_f32))
    if relu:
        pn = jnp.maximum(pn, 0.0)
        un = jnp.maximum(un, 0.0)
    op[...] = pn
    ou[...] = un


def _tc_update(relu, sp, su, cnt, u, p, wlb, blb, wrb, wlr, blr, wrr):
    spec_c = pl.BlockSpec((2, _BLK, 16), lambda i: (0, i, 0))
    spec_x = pl.BlockSpec((_BLK, D), lambda i: (i, 0))
    spec_w = pl.BlockSpec((D, D), lambda i: (0, 0))
    spec_b = pl.BlockSpec((1, D), lambda i: (0, 0))
    return pl.pallas_call(
        functools.partial(_update_body, relu),
        grid=(_GRID,),
        in_specs=[spec_x, spec_x, spec_c, spec_x, spec_x,
                  spec_w, spec_b, spec_w, spec_w, spec_b, spec_w],
        out_specs=[spec_x, spec_x],
        out_shape=[jax.ShapeDtypeStruct((N_PAD, D), _f32)] * 2,
    )(sp, su, cnt, u, p, wlb, blb.reshape(1, D), wrb,
      wlr, blr.reshape(1, D), wrr)


@functools.partial(
    pl.kernel,
    out_type=jax.ShapeDtypeStruct((2, N_PAD, 16), _f32),
    mesh=_mesh,
    compiler_params=_sc_params,
    scratch_types=[
        pltpu.VMEM_SHARED((N_PAD, 16), _f32),
        pltpu.VMEM((CHUNK,), jnp.int32),
        pltpu.VMEM((CHUNK, 16), _f32),
    ],
)
def _sc_counts(esrc, edst, zeros16, out, acc, dbuf, ones):
    c = lax.axis_index("c")
    s = lax.axis_index("s")

    @pl.loop(0, CHUNK)
    def _(i):
        ones[i, :] = jnp.ones((16,), _f32)

    pltpu.sync_copy(zeros16.at[pl.ds(s * ROWS_PER_TILE, ROWS_PER_TILE)],
                    acc.at[pl.ds(s * ROWS_PER_TILE, ROWS_PER_TILE)])
    plsc.subcore_barrier()

    @pl.loop(0, CHUNKS_PER_TILE)
    def _(ci):
        base = (s * CHUNKS_PER_TILE + ci) * CHUNK

        @pl.when(c == 0)
        def _():
            pltpu.sync_copy(edst.at[pl.ds(base, CHUNK)], dbuf)

        @pl.when(c == 1)
        def _():
            pltpu.sync_copy(esrc.at[pl.ds(base, CHUNK)], dbuf)

        pltpu.sync_copy(ones, acc.at[dbuf], add=True)

    plsc.subcore_barrier()
    pltpu.sync_copy(acc.at[pl.ds(s * ROWS_PER_TILE, ROWS_PER_TILE)],
                    out.at[c, pl.ds(s * ROWS_PER_TILE, ROWS_PER_TILE)])


@functools.partial(
    pl.kernel,
    out_type=[jax.ShapeDtypeStruct((N_PAD, D), _f32)] * 2,
    mesh=_mesh,
    compiler_params=_sc_params,
    scratch_types=[
        pltpu.VMEM_SHARED((N_PAD, 32), _f32),
        pltpu.VMEM((2, G, CHUNK), jnp.int32),
        pltpu.VMEM((2, G, CHUNK), jnp.int32),
        pltpu.VMEM((2, G, CHUNK), jnp.int32),
        pltpu.VMEM((2, G, CHUNK, 32), _f32),
        pltpu.SemaphoreType.DMA((2,)),
        pltpu.SemaphoreType.DMA((2,)),
        pltpu.SemaphoreType.DMA((2,)),
    ],
)
def _sc_segsum(u4, p4, esrc, edst, zeros32, sp_out, su_out,
               acc, sbuf, gbuf, dbuf, rbuf, sem_i, sem_g, sem_a):
    c = lax.axis_index("c")
    s = lax.axis_index("s")

    for d in range(2):
        tab = u4 if d == 0 else p4
        src_h = esrc if d == 0 else edst
        dst_h = edst if d == 0 else esrc
        out_h = sp_out if d == 0 else su_out
        for r in range(2):
            q = r * 2 + c
            pltpu.sync_copy(zeros32.at[pl.ds(s * ROWS_PER_TILE, ROWS_PER_TILE)],
                            acc.at[pl.ds(s * ROWS_PER_TILE, ROWS_PER_TILE)])
            plsc.subcore_barrier()
            tile_base = s * CHUNKS_PER_TILE * CHUNK

            def idx_start(gi, ph, j):
                b = tile_base + (gi * G + j) * CHUNK
                pltpu.async_copy(src_h.at[pl.ds(b, CHUNK)], sbuf.at[ph, j],
                                 sem_i.at[ph])
                pltpu.async_copy(dst_h.at[pl.ds(b, CHUNK)], dbuf.at[ph, j],
                                 sem_i.at[ph])

            def idx_wait(ph, j):
                pltpu.make_async_copy(src_h.at[pl.ds(0, CHUNK)],
                                      sbuf.at[ph, j], sem_i.at[ph]).wait()
                pltpu.make_async_copy(dst_h.at[pl.ds(0, CHUNK)],
                                      dbuf.at[ph, j], sem_i.at[ph]).wait()

            def add_wait(ph, j):
                pltpu.make_async_copy(rbuf.at[ph, j], acc.at[dbuf.at[ph, j]],
                                      sem_a.at[ph]).wait()

            def group_body(gi, ph):
                for j in range(G):
                    idx_wait(ph, j)
                for j in range(G):
                    @pl.loop(0, CHUNK, step=16)
                    def _(i):
                        gbuf[ph, j, pl.ds(i, 16)] = (
                            sbuf[ph, j, pl.ds(i, 16)] * 4 + q)
                @pl.when(gi >= 2)
                def _():
                    for j in range(G):
                        add_wait(ph, j)
                for j in range(G):
                    pltpu.async_copy(tab.at[gbuf.at[ph, j]], rbuf.at[ph, j],
                                     sem_g.at[ph])
                @pl.when(gi + 1 < GROUPS)
                def _():
                    for j in range(G):
                        idx_start(gi + 1, 1 - ph, j)
                for j in range(G):
                    pltpu.make_async_copy(tab.at[gbuf.at[ph, j]],
                                          rbuf.at[ph, j], sem_g.at[ph]).wait()
                for j in range(G):
                    pltpu.async_copy(rbuf.at[ph, j], acc.at[dbuf.at[ph, j]],
                                     sem_a.at[ph], add=True)

            for j in range(G):
                idx_start(0, 0, j)

            @pl.loop(0, HALF_GROUPS)
            def _(h):
                group_body(2 * h, 0)
                group_body(2 * h + 1, 1)

            for ph in range(2):
                for j in range(G):
                    add_wait(ph, j)
            plsc.subcore_barrier()
            pltpu.sync_copy(acc.at[pl.ds(s * ROWS_PER_TILE, ROWS_PER_TILE)],
                            out_h.at[pl.ds(s * ROWS_PER_TILE, ROWS_PER_TILE),
                                     pl.ds(q * 32, 32)])


@functools.partial(
    pl.kernel,
    out_type=[jax.ShapeDtypeStruct((L_PAD, D), _f32)] * 2,
    mesh=_mesh,
    compiler_params=_sc_params,
    scratch_types=[
        pltpu.VMEM((LCHUNK,), jnp.int32),
        pltpu.VMEM((LCHUNK,), jnp.int32),
        pltpu.VMEM((LCHUNK, D), _f32),
        pltpu.VMEM((LCHUNK, D), _f32),
    ],
)
def _sc_gather_pairs(u, p, eli0, eli1, ue_out, pe_out, i0, i1, ubuf, pbuf):
    c = lax.axis_index("c")
    s = lax.axis_index("s")
    w = c * 16 + s

    @pl.loop(0, LCHUNKS)
    def _(k):
        base = w * ROWS_PER_TILE + k * LCHUNK
        pltpu.sync_copy(eli0.at[pl.ds(base, LCHUNK)], i0)
        pltpu.sync_copy(eli1.at[pl.ds(base, LCHUNK)], i1)
        pltpu.sync_copy(u.at[i0], ubuf)
        pltpu.sync_copy(p.at[i1], pbuf)
        pltpu.sync_copy(ubuf, ue_out.at[pl.ds(base, LCHUNK)])
        pltpu.sync_copy(pbuf, pe_out.at[pl.ds(base, LCHUNK)])


def _dot_body(a, b, o):
    o[...] = jnp.sum(a[...] * b[...], axis=1, keepdims=True)


def _tc_dot(ue, pe):
    spec_x = pl.BlockSpec((_BLK, D), lambda i: (i, 0))
    return pl.pallas_call(
        _dot_body,
        grid=(L_PAD // _BLK,),
        in_specs=[spec_x, spec_x],
        out_specs=pl.BlockSpec((_BLK, 1), lambda i: (i, 0)),
        out_shape=jax.ShapeDtypeStruct((L_PAD, 1), _f32),
    )(ue, pe)


def kernel(user_x, prod_x, edge_index, edge_label_index,
           W_user, b_user, W_prod, b_prod,
           W_l1_buy, b_l1_buy, W_r1_buy, W_l1_rev, b_l1_rev, W_r1_rev,
           W_l2_buy, b_l2_buy, W_r2_buy, W_l2_rev, b_l2_rev, W_r2_rev,
           W_l3_buy, b_l3_buy, W_r3_buy, W_l3_rev, b_l3_rev, W_r3_rev):
    ux = jnp.pad(user_x, ((0, N_PAD - N), (0, 0)))
    px = jnp.pad(prod_x, ((0, N_PAD - N), (0, 0)))

    npad = E_PAD - E
    fill = jnp.arange(npad, dtype=jnp.int32)
    esrc = jnp.concatenate([edge_index[0], N + fill % (N_PAD - N)])
    edst = jnp.concatenate([edge_index[1], N + (fill * 7 + 3) % (N_PAD - N)])

    zeros32 = jnp.zeros((N_PAD, 32), _f32)
    zeros16 = jnp.zeros((N_PAD, 16), _f32)

    cnt = _sc_counts(esrc, edst, zeros16)

    u, p = _tc_init(ux, W_user, b_user, px, W_prod, b_prod)

    layers = [
        (W_l1_buy, b_l1_buy, W_r1_buy, W_l1_rev, b_l1_rev, W_r1_rev),
        (W_l2_buy, b_l2_buy, W_r2_buy, W_l2_rev, b_l2_rev, W_r2_rev),
        (W_l3_buy, b_l3_buy, W_r3_buy, W_l3_rev, b_l3_rev, W_r3_rev),
    ]
    for i, (wlb, blb, wrb, wlr, blr, wrr) in enumerate(layers):
        sp, su = _sc_segsum(u.reshape(4 * N_PAD, 32), p.reshape(4 * N_PAD, 32),
                            esrc, edst, zeros32)
        p, u = _tc_update(i < 2, sp, su, cnt, u, p,
                          wlb, blb, wrb, wlr, blr, wrr)

    lpad = L_PAD - L
    lfill = jnp.arange(lpad, dtype=jnp.int32) % N
    eli0 = jnp.concatenate([edge_label_index[0], lfill])
    eli1 = jnp.concatenate([edge_label_index[1], lfill])
    ue, pe = _sc_gather_pairs(u, p, eli0, eli1)
    pred = _tc_dot(ue, pe)
    return pred[:L, 0]


# speedup vs baseline: 7.6410x; 1.0622x over previous
import functools

import jax
import jax.numpy as jnp
from jax import lax
from jax.experimental import pallas as pl
from jax.experimental.pallas import tpu as pltpu
from jax.experimental.pallas import tpu_sc as plsc

N = 50000
D = 128
E = 625000
L = 100000

N_PAD = 50176
ROWS_PER_TILE = N_PAD // 16
CHUNK = 128
CHUNKS_PER_TILE = 312
OCTS = CHUNKS_PER_TILE // 8
E_PAD = 16 * CHUNKS_PER_TILE * CHUNK
L_PAD = 100352
LCHUNK = 112
LCHUNKS = 28

_BLK = 512
_GRID = N_PAD // _BLK

_mesh = plsc.VectorSubcoreMesh(core_axis_name="c", subcore_axis_name="s")
_f32 = jnp.float32
_sc_params = pltpu.CompilerParams(use_tc_tiling_on_sc=False,
                                  needs_layout_passes=False)


def _init_body(ux, wu, bu, px, wp, bp, ou, op):
    ou[...] = jnp.dot(ux[...], wu[...], preferred_element_type=_f32) + bu[...]
    op[...] = jnp.dot(px[...], wp[...], preferred_element_type=_f32) + bp[...]


def _tc_init(ux, wu, bu, px, wp, bp):
    spec_x = pl.BlockSpec((_BLK, D), lambda i: (i, 0))
    spec_w = pl.BlockSpec((D, D), lambda i: (0, 0))
    spec_b = pl.BlockSpec((1, D), lambda i: (0, 0))
    return pl.pallas_call(
        _init_body,
        grid=(_GRID,),
        in_specs=[spec_x, spec_w, spec_b, spec_x, spec_w, spec_b],
        out_specs=[spec_x, spec_x],
        out_shape=[jax.ShapeDtypeStruct((N_PAD, D), _f32)] * 2,
    )(ux, wu, bu.reshape(1, D), px, wp, bp.reshape(1, D))


def _update_body(relu, sp, su, cnt, u, p, wlb, blb, wrb, wlr, blr, wrr,
                 op, ou):
    mean_p = sp[...]
    mean_u = su[...]
    c = cnt[...]
    rp = 1.0 / jnp.maximum(c[0, :, 0:1], 1.0)
    ru = 1.0 / jnp.maximum(c[1, :, 0:1], 1.0)
    pn = (jnp.dot(mean_p * rp, wlb[...], preferred_element_type=_f32) + blb[...]
          + jnp.dot(p[...], wrb[...], preferred_element_type=_f32))
    un = (jnp.dot(mean_u * ru, wlr[...], preferred_element_type=_f32) + blr[...]
          + jnp.dot(u[...], wrr[...], preferred_element_type=_f32))
    if relu:
        pn = jnp.maximum(pn, 0.0)
        un = jnp.maximum(un, 0.0)
    op[...] = pn
    ou[...] = un


def _tc_update(relu, sp, su, cnt, u, p, wlb, blb, wrb, wlr, blr, wrr):
    spec_c = pl.BlockSpec((2, _BLK, 16), lambda i: (0, i, 0))
    spec_x = pl.BlockSpec((_BLK, D), lambda i: (i, 0))
    spec_w = pl.BlockSpec((D, D), lambda i: (0, 0))
    spec_b = pl.BlockSpec((1, D), lambda i: (0, 0))
    return pl.pallas_call(
        functools.partial(_update_body, relu),
        grid=(_GRID,),
        in_specs=[spec_x, spec_x, spec_c, spec_x, spec_x,
                  spec_w, spec_b, spec_w, spec_w, spec_b, spec_w],
        out_specs=[spec_x, spec_x],
        out_shape=[jax.ShapeDtypeStruct((N_PAD, D), _f32)] * 2,
    )(sp, su, cnt, u, p, wlb, blb.reshape(1, D), wrb,
      wlr, blr.reshape(1, D), wrr)


@functools.partial(
    pl.kernel,
    out_type=jax.ShapeDtypeStruct((2, N_PAD, 16), _f32),
    mesh=_mesh,
    compiler_params=_sc_params,
    scratch_types=[
        pltpu.VMEM_SHARED((N_PAD, 16), _f32),
        pltpu.VMEM((CHUNK,), jnp.int32),
        pltpu.VMEM((CHUNK, 16), _f32),
    ],
)
def _sc_counts(esrc, edst, zeros16, out, acc, dbuf, ones):
    c = lax.axis_index("c")
    s = lax.axis_index("s")

    @pl.loop(0, CHUNK)
    def _(i):
        ones[i, :] = jnp.ones((16,), _f32)

    pltpu.sync_copy(zeros16.at[pl.ds(s * ROWS_PER_TILE, ROWS_PER_TILE)],
                    acc.at[pl.ds(s * ROWS_PER_TILE, ROWS_PER_TILE)])
    plsc.subcore_barrier()

    @pl.loop(0, CHUNKS_PER_TILE)
    def _(ci):
        base = (s * CHUNKS_PER_TILE + ci) * CHUNK

        @pl.when(c == 0)
        def _():
            pltpu.sync_copy(edst.at[pl.ds(base, CHUNK)], dbuf)

        @pl.when(c == 1)
        def _():
            pltpu.sync_copy(esrc.at[pl.ds(base, CHUNK)], dbuf)

        pltpu.sync_copy(ones, acc.at[dbuf], add=True)

    plsc.subcore_barrier()
    pltpu.sync_copy(acc.at[pl.ds(s * ROWS_PER_TILE, ROWS_PER_TILE)],
                    out.at[c, pl.ds(s * ROWS_PER_TILE, ROWS_PER_TILE)])


@functools.partial(
    pl.kernel,
    out_type=[jax.ShapeDtypeStruct((N_PAD, D), _f32)] * 2,
    mesh=_mesh,
    compiler_params=_sc_params,
    scratch_types=[
        pltpu.VMEM_SHARED((N_PAD, 32), _f32),
        pltpu.VMEM((4, CHUNK), jnp.int32),
        pltpu.VMEM((4, CHUNK), jnp.int32),
        pltpu.VMEM((8, CHUNK), jnp.int32),
        pltpu.VMEM((4, CHUNK, 32), _f32),
        pltpu.SemaphoreType.DMA((4,)),
        pltpu.SemaphoreType.DMA((4,)),
        pltpu.SemaphoreType.DMA((4,)),
    ],
)
def _sc_segsum(u4, p4, esrc, edst, zeros32, sp_out, su_out,
               acc, sbuf, gbuf, dbuf, rbuf, sem_i, sem_g, sem_a):
    c = lax.axis_index("c")
    s = lax.axis_index("s")
    NCH = CHUNKS_PER_TILE

    for d in range(2):
        tab = u4 if d == 0 else p4
        src_h = esrc if d == 0 else edst
        dst_h = edst if d == 0 else esrc
        out_h = sp_out if d == 0 else su_out
        for r in range(2):
            q = r * 2 + c
            pltpu.sync_copy(zeros32.at[pl.ds(s * ROWS_PER_TILE, ROWS_PER_TILE)],
                            acc.at[pl.ds(s * ROWS_PER_TILE, ROWS_PER_TILE)])
            plsc.subcore_barrier()
            tile_base = s * CHUNKS_PER_TILE * CHUNK

            def idx_start(g, s4, s8):
                b = tile_base + g * CHUNK
                pltpu.async_copy(src_h.at[pl.ds(b, CHUNK)], sbuf.at[s4],
                                 sem_i.at[s4])
                pltpu.async_copy(dst_h.at[pl.ds(b, CHUNK)], dbuf.at[s8],
                                 sem_i.at[s4])

            def idx_wait(s4, s8):
                pltpu.make_async_copy(src_h.at[pl.ds(0, CHUNK)],
                                      sbuf.at[s4], sem_i.at[s4]).wait()
                pltpu.make_async_copy(dst_h.at[pl.ds(0, CHUNK)],
                                      dbuf.at[s8], sem_i.at[s4]).wait()

            def transform(s4):
                @pl.loop(0, CHUNK, step=16)
                def _(i):
                    gbuf[s4, pl.ds(i, 16)] = sbuf[s4, pl.ds(i, 16)] * 4 + q

            def gather_start(s4):
                pltpu.async_copy(tab.at[gbuf.at[s4]], rbuf.at[s4],
                                 sem_g.at[s4])

            def gather_wait(s4):
                pltpu.make_async_copy(tab.at[gbuf.at[s4]], rbuf.at[s4],
                                      sem_g.at[s4]).wait()

            def add_start(s4, s8):
                pltpu.async_copy(rbuf.at[s4], acc.at[dbuf.at[s8]],
                                 sem_a.at[s4], add=True)

            def add_wait(s4, s8):
                pltpu.make_async_copy(rbuf.at[s4], acc.at[dbuf.at[s8]],
                                      sem_a.at[s4]).wait()

            idx_start(0, 0, 0)
            idx_start(1, 1, 1)
            idx_start(2, 2, 2)
            idx_wait(0, 0)
            transform(0)
            idx_wait(1, 1)
            transform(1)
            gather_start(0)

            @pl.loop(0, OCTS)
            def _(h):
                for j in range(8):
                    g = 8 * h + j
                    s4 = j % 4
                    if j + 1 < 8:
                        gather_start((j + 1) % 4)
                    else:
                        @pl.when(g + 1 < NCH)
                        def _():
                            gather_start((j + 1) % 4)
                    @pl.when(g + 3 < NCH)
                    def _():
                        idx_start(g + 3, (j + 3) % 4, (j + 3) % 8)
                    @pl.when(g + 2 < NCH)
                    def _():
                        idx_wait((j + 2) % 4, (j + 2) % 8)
                        transform((j + 2) % 4)
                    gather_wait(s4)
                    add_start(s4, j)
                    @pl.when(g >= 2)
                    def _():
                        add_wait((j + 2) % 4, (j + 6) % 8)

            add_wait(2, 6)
            add_wait(3, 7)
            plsc.subcore_barrier()
            pltpu.sync_copy(acc.at[pl.ds(s * ROWS_PER_TILE, ROWS_PER_TILE)],
                            out_h.at[pl.ds(s * ROWS_PER_TILE, ROWS_PER_TILE),
                                     pl.ds(q * 32, 32)])


@functools.partial(
    pl.kernel,
    out_type=[jax.ShapeDtypeStruct((L_PAD, D), _f32)] * 2,
    mesh=_mesh,
    compiler_params=_sc_params,
    scratch_types=[
        pltpu.VMEM((LCHUNK,), jnp.int32),
        pltpu.VMEM((LCHUNK,), jnp.int32),
        pltpu.VMEM((LCHUNK, D), _f32),
        pltpu.VMEM((LCHUNK, D), _f32),
    ],
)
def _sc_gather_pairs(u, p, eli0, eli1, ue_out, pe_out, i0, i1, ubuf, pbuf):
    c = lax.axis_index("c")
    s = lax.axis_index("s")
    w = c * 16 + s

    @pl.loop(0, LCHUNKS)
    def _(k):
        base = w * ROWS_PER_TILE + k * LCHUNK
        pltpu.sync_copy(eli0.at[pl.ds(base, LCHUNK)], i0)
        pltpu.sync_copy(eli1.at[pl.ds(base, LCHUNK)], i1)
        pltpu.sync_copy(u.at[i0], ubuf)
        pltpu.sync_copy(p.at[i1], pbuf)
        pltpu.sync_copy(ubuf, ue_out.at[pl.ds(base, LCHUNK)])
        pltpu.sync_copy(pbuf, pe_out.at[pl.ds(base, LCHUNK)])


def _dot_body(a, b, o):
    o[...] = jnp.sum(a[...] * b[...], axis=1, keepdims=True)


def _tc_dot(ue, pe):
    spec_x = pl.BlockSpec((_BLK, D), lambda i: (i, 0))
    return pl.pallas_call(
        _dot_body,
        grid=(L_PAD // _BLK,),
        in_specs=[spec_x, spec_x],
        out_specs=pl.BlockSpec((_BLK, 1), lambda i: (i, 0)),
        out_shape=jax.ShapeDtypeStruct((L_PAD, 1), _f32),
    )(ue, pe)


def kernel(user_x, prod_x, edge_index, edge_label_index,
           W_user, b_user, W_prod, b_prod,
           W_l1_buy, b_l1_buy, W_r1_buy, W_l1_rev, b_l1_rev, W_r1_rev,
           W_l2_buy, b_l2_buy, W_r2_buy, W_l2_rev, b_l2_rev, W_r2_rev,
           W_l3_buy, b_l3_buy, W_r3_buy, W_l3_rev, b_l3_rev, W_r3_rev):
    ux = jnp.pad(user_x, ((0, N_PAD - N), (0, 0)))
    px = jnp.pad(prod_x, ((0, N_PAD - N), (0, 0)))

    npad = E_PAD - E
    fill = jnp.arange(npad, dtype=jnp.int32)
    esrc = jnp.concatenate([edge_index[0], N + fill % (N_PAD - N)])
    edst = jnp.concatenate([edge_index[1], N + (fill * 7 + 3) % (N_PAD - N)])

    zeros32 = jnp.zeros((N_PAD, 32), _f32)
    zeros16 = jnp.zeros((N_PAD, 16), _f32)

    cnt = _sc_counts(esrc, edst, zeros16)

    u, p = _tc_init(ux, W_user, b_user, px, W_prod, b_prod)

    layers = [
        (W_l1_buy, b_l1_buy, W_r1_buy, W_l1_rev, b_l1_rev, W_r1_rev),
        (W_l2_buy, b_l2_buy, W_r2_buy, W_l2_rev, b_l2_rev, W_r2_rev),
        (W_l3_buy, b_l3_buy, W_r3_buy, W_l3_rev, b_l3_rev, W_r3_rev),
    ]
    for i, (wlb, blb, wrb, wlr, blr, wrr) in enumerate(layers):
        sp, su = _sc_segsum(u.reshape(4 * N_PAD, 32), p.reshape(4 * N_PAD, 32),
                            esrc, edst, zeros32)
        p, u = _tc_update(i < 2, sp, su, cnt, u, p,
                          wlb, blb, wrb, wlr, blr, wrr)

    lpad = L_PAD - L
    lfill = jnp.arange(lpad, dtype=jnp.int32) % N
    eli0 = jnp.concatenate([edge_label_index[0], lfill])
    eli1 = jnp.concatenate([edge_label_index[1], lfill])
    ue, pe = _sc_gather_pairs(u, p, eli0, eli1)
    pred = _tc_dot(ue, pe)
    return pred[:L, 0]


# speedup vs baseline: 7.8613x; 1.0288x over previous
import functools

import jax
import jax.numpy as jnp
from jax import lax
from jax.experimental import pallas as pl
from jax.experimental.pallas import tpu as pltpu
from jax.experimental.pallas import tpu_sc as plsc

N = 50000
D = 128
E = 625000
L = 100000

N_PAD = 50176
ROWS_PER_TILE = N_PAD // 16
CHUNK = 128
CHUNKS_PER_TILE = 312
OCTS = CHUNKS_PER_TILE // 8
E_PAD = 16 * CHUNKS_PER_TILE * CHUNK
CCHUNK = 256
CCHUNKS_PER_TILE = 156
L_PAD = 100352
LCHUNK = 112
LCHUNKS = 28

_BLK = 512
_GRID = N_PAD // _BLK

_mesh = plsc.VectorSubcoreMesh(core_axis_name="c", subcore_axis_name="s")
_f32 = jnp.float32
_sc_params = pltpu.CompilerParams(use_tc_tiling_on_sc=False,
                                  needs_layout_passes=False)


def _init_body(ux, wu, bu, px, wp, bp, ou, op):
    ou[...] = jnp.dot(ux[...], wu[...], preferred_element_type=_f32) + bu[...]
    op[...] = jnp.dot(px[...], wp[...], preferred_element_type=_f32) + bp[...]


def _tc_init(ux, wu, bu, px, wp, bp):
    spec_x = pl.BlockSpec((_BLK, D), lambda i: (i, 0))
    spec_w = pl.BlockSpec((D, D), lambda i: (0, 0))
    spec_b = pl.BlockSpec((1, D), lambda i: (0, 0))
    return pl.pallas_call(
        _init_body,
        grid=(_GRID,),
        in_specs=[spec_x, spec_w, spec_b, spec_x, spec_w, spec_b],
        out_specs=[spec_x, spec_x],
        out_shape=[jax.ShapeDtypeStruct((N_PAD, D), _f32)] * 2,
    )(ux, wu, bu.reshape(1, D), px, wp, bp.reshape(1, D))


def _update_body(relu, sp, su, cnt, u, p, wlb, blb, wrb, wlr, blr, wrr,
                 op, ou):
    mean_p = sp[...]
    mean_u = su[...]
    c = cnt[...]
    rp = 1.0 / jnp.maximum(c[0, :, 0:1], 1.0)
    ru = 1.0 / jnp.maximum(c[1, :, 0:1], 1.0)
    pn = (jnp.dot(mean_p * rp, wlb[...], preferred_element_type=_f32) + blb[...]
          + jnp.dot(p[...], wrb[...], preferred_element_type=_f32))
    un = (jnp.dot(mean_u * ru, wlr[...], preferred_element_type=_f32) + blr[...]
          + jnp.dot(u[...], wrr[...], preferred_element_type=_f32))
    if relu:
        pn = jnp.maximum(pn, 0.0)
        un = jnp.maximum(un, 0.0)
    op[...] = pn
    ou[...] = un


def _tc_update(relu, sp, su, cnt, u, p, wlb, blb, wrb, wlr, blr, wrr):
    spec_c = pl.BlockSpec((2, _BLK, 16), lambda i: (0, i, 0))
    spec_x = pl.BlockSpec((_BLK, D), lambda i: (i, 0))
    spec_w = pl.BlockSpec((D, D), lambda i: (0, 0))
    spec_b = pl.BlockSpec((1, D), lambda i: (0, 0))
    return pl.pallas_call(
        functools.partial(_update_body, relu),
        grid=(_GRID,),
        in_specs=[spec_x, spec_x, spec_c, spec_x, spec_x,
                  spec_w, spec_b, spec_w, spec_w, spec_b, spec_w],
        out_specs=[spec_x, spec_x],
        out_shape=[jax.ShapeDtypeStruct((N_PAD, D), _f32)] * 2,
    )(sp, su, cnt, u, p, wlb, blb.reshape(1, D), wrb,
      wlr, blr.reshape(1, D), wrr)


@functools.partial(
    pl.kernel,
    out_type=jax.ShapeDtypeStruct((2, N_PAD, 16), _f32),
    mesh=_mesh,
    compiler_params=_sc_params,
    scratch_types=[
        pltpu.VMEM_SHARED((N_PAD, 16), _f32),
        pltpu.VMEM((CCHUNK,), jnp.int32),
        pltpu.VMEM((CCHUNK, 16), _f32),
    ],
)
def _sc_counts(esrc, edst, zeros16, out, acc, dbuf, ones):
    c = lax.axis_index("c")
    s = lax.axis_index("s")

    @pl.loop(0, CCHUNK)
    def _(i):
        ones[i, :] = jnp.ones((16,), _f32)

    pltpu.sync_copy(zeros16.at[pl.ds(s * ROWS_PER_TILE, ROWS_PER_TILE)],
                    acc.at[pl.ds(s * ROWS_PER_TILE, ROWS_PER_TILE)])
    plsc.subcore_barrier()

    @pl.loop(0, CCHUNKS_PER_TILE)
    def _(ci):
        base = (s * CCHUNKS_PER_TILE + ci) * CCHUNK

        @pl.when(c == 0)
        def _():
            pltpu.sync_copy(edst.at[pl.ds(base, CCHUNK)], dbuf)

        @pl.when(c == 1)
        def _():
            pltpu.sync_copy(esrc.at[pl.ds(base, CCHUNK)], dbuf)

        pltpu.sync_copy(ones, acc.at[dbuf], add=True)

    plsc.subcore_barrier()
    pltpu.sync_copy(acc.at[pl.ds(s * ROWS_PER_TILE, ROWS_PER_TILE)],
                    out.at[c, pl.ds(s * ROWS_PER_TILE, ROWS_PER_TILE)])


@functools.partial(
    pl.kernel,
    out_type=[jax.ShapeDtypeStruct((N_PAD, D), _f32)] * 2,
    mesh=_mesh,
    compiler_params=_sc_params,
    scratch_types=[
        pltpu.VMEM_SHARED((N_PAD, 32), _f32),
        pltpu.VMEM((4, CHUNK), jnp.int32),
        pltpu.VMEM((4, CHUNK), jnp.int32),
        pltpu.VMEM((8, CHUNK), jnp.int32),
        pltpu.VMEM((4, CHUNK, 32), _f32),
        pltpu.SemaphoreType.DMA((4,)),
        pltpu.SemaphoreType.DMA((4,)),
        pltpu.SemaphoreType.DMA((4,)),
    ],
)
def _sc_segsum(u4, p4, esrc, edst, zeros32, sp_out, su_out,
               acc, sbuf, gbuf, dbuf, rbuf, sem_i, sem_g, sem_a):
    c = lax.axis_index("c")
    s = lax.axis_index("s")
    NCH = CHUNKS_PER_TILE

    for d in range(2):
        tab = u4 if d == 0 else p4
        src_h = esrc if d == 0 else edst
        dst_h = edst if d == 0 else esrc
        out_h = sp_out if d == 0 else su_out
        for r in range(2):
            q = r * 2 + c
            pltpu.sync_copy(zeros32.at[pl.ds(s * ROWS_PER_TILE, ROWS_PER_TILE)],
                            acc.at[pl.ds(s * ROWS_PER_TILE, ROWS_PER_TILE)])
            plsc.subcore_barrier()
            tile_base = s * CHUNKS_PER_TILE * CHUNK

            def idx_start(g, s4, s8):
                b = tile_base + g * CHUNK
                pltpu.async_copy(src_h.at[pl.ds(b, CHUNK)], sbuf.at[s4],
                                 sem_i.at[s4])
                pltpu.async_copy(dst_h.at[pl.ds(b, CHUNK)], dbuf.at[s8],
                                 sem_i.at[s4])

            def idx_wait(s4, s8):
                pltpu.make_async_copy(src_h.at[pl.ds(0, CHUNK)],
                                      sbuf.at[s4], sem_i.at[s4]).wait()
                pltpu.make_async_copy(dst_h.at[pl.ds(0, CHUNK)],
                                      dbuf.at[s8], sem_i.at[s4]).wait()

            def transform(s4):
                @pl.loop(0, CHUNK, step=16)
                def _(i):
                    gbuf[s4, pl.ds(i, 16)] = sbuf[s4, pl.ds(i, 16)] * 4 + q

            def gather_start(s4):
                pltpu.async_copy(tab.at[gbuf.at[s4]], rbuf.at[s4],
                                 sem_g.at[s4])

            def gather_wait(s4):
                pltpu.make_async_copy(tab.at[gbuf.at[s4]], rbuf.at[s4],
                                      sem_g.at[s4]).wait()

            def add_start(s4, s8):
                pltpu.async_copy(rbuf.at[s4], acc.at[dbuf.at[s8]],
                                 sem_a.at[s4], add=True)

            def add_wait(s4, s8):
                pltpu.make_async_copy(rbuf.at[s4], acc.at[dbuf.at[s8]],
                                      sem_a.at[s4]).wait()

            idx_start(0, 0, 0)
            idx_start(1, 1, 1)
            idx_start(2, 2, 2)
            idx_wait(0, 0)
            transform(0)
            idx_wait(1, 1)
            transform(1)
            gather_start(0)

            @pl.loop(0, OCTS)
            def _(h):
                for j in range(8):
                    g = 8 * h + j
                    s4 = j % 4
                    if j + 1 < 8:
                        gather_start((j + 1) % 4)
                    else:
                        @pl.when(g + 1 < NCH)
                        def _():
                            gather_start((j + 1) % 4)
                    @pl.when(g + 3 < NCH)
                    def _():
                        idx_start(g + 3, (j + 3) % 4, (j + 3) % 8)
                    @pl.when(g + 2 < NCH)
                    def _():
                        idx_wait((j + 2) % 4, (j + 2) % 8)
                        transform((j + 2) % 4)
                    gather_wait(s4)
                    add_start(s4, j)
                    @pl.when(g >= 2)
                    def _():
                        add_wait((j + 2) % 4, (j + 6) % 8)

            add_wait(2, 6)
            add_wait(3, 7)
            plsc.subcore_barrier()
            pltpu.sync_copy(acc.at[pl.ds(s * ROWS_PER_TILE, ROWS_PER_TILE)],
                            out_h.at[pl.ds(s * ROWS_PER_TILE, ROWS_PER_TILE),
                                     pl.ds(q * 32, 32)])


@functools.partial(
    pl.kernel,
    out_type=[jax.ShapeDtypeStruct((L_PAD, D), _f32)] * 2,
    mesh=_mesh,
    compiler_params=_sc_params,
    scratch_types=[
        pltpu.VMEM((LCHUNK,), jnp.int32),
        pltpu.VMEM((LCHUNK,), jnp.int32),
        pltpu.VMEM((LCHUNK, D), _f32),
        pltpu.VMEM((LCHUNK, D), _f32),
    ],
)
def _sc_gather_pairs(u, p, eli0, eli1, ue_out, pe_out, i0, i1, ubuf, pbuf):
    c = lax.axis_index("c")
    s = lax.axis_index("s")
    w = c * 16 + s

    @pl.loop(0, LCHUNKS)
    def _(k):
        base = w * ROWS_PER_TILE + k * LCHUNK
        pltpu.sync_copy(eli0.at[pl.ds(base, LCHUNK)], i0)
        pltpu.sync_copy(eli1.at[pl.ds(base, LCHUNK)], i1)
        pltpu.sync_copy(u.at[i0], ubuf)
        pltpu.sync_copy(p.at[i1], pbuf)
        pltpu.sync_copy(ubuf, ue_out.at[pl.ds(base, LCHUNK)])
        pltpu.sync_copy(pbuf, pe_out.at[pl.ds(base, LCHUNK)])


def _dot_body(a, b, o):
    o[...] = jnp.sum(a[...] * b[...], axis=1, keepdims=True)


def _tc_dot(ue, pe):
    spec_x = pl.BlockSpec((_BLK, D), lambda i: (i, 0))
    return pl.pallas_call(
        _dot_body,
        grid=(L_PAD // _BLK,),
        in_specs=[spec_x, spec_x],
        out_specs=pl.BlockSpec((_BLK, 1), lambda i: (i, 0)),
        out_shape=jax.ShapeDtypeStruct((L_PAD, 1), _f32),
    )(ue, pe)


def kernel(user_x, prod_x, edge_index, edge_label_index,
           W_user, b_user, W_prod, b_prod,
           W_l1_buy, b_l1_buy, W_r1_buy, W_l1_rev, b_l1_rev, W_r1_rev,
           W_l2_buy, b_l2_buy, W_r2_buy, W_l2_rev, b_l2_rev, W_r2_rev,
           W_l3_buy, b_l3_buy, W_r3_buy, W_l3_rev, b_l3_rev, W_r3_rev):
    ux = jnp.pad(user_x, ((0, N_PAD - N), (0, 0)))
    px = jnp.pad(prod_x, ((0, N_PAD - N), (0, 0)))

    npad = E_PAD - E
    fill = jnp.arange(npad, dtype=jnp.int32)
    esrc = jnp.concatenate([edge_index[0], N + fill % (N_PAD - N)])
    edst = jnp.concatenate([edge_index[1], N + (fill * 7 + 3) % (N_PAD - N)])

    zeros32 = jnp.zeros((N_PAD, 32), _f32)
    zeros16 = jnp.zeros((N_PAD, 16), _f32)

    cnt = _sc_counts(esrc, edst, zeros16)

    u, p = _tc_init(ux, W_user, b_user, px, W_prod, b_prod)

    layers = [
        (W_l1_buy, b_l1_buy, W_r1_buy, W_l1_rev, b_l1_rev, W_r1_rev),
        (W_l2_buy, b_l2_buy, W_r2_buy, W_l2_rev, b_l2_rev, W_r2_rev),
        (W_l3_buy, b_l3_buy, W_r3_buy, W_l3_rev, b_l3_rev, W_r3_rev),
    ]
    for i, (wlb, blb, wrb, wlr, blr, wrr) in enumerate(layers):
        sp, su = _sc_segsum(u.reshape(4 * N_PAD, 32), p.reshape(4 * N_PAD, 32),
                            esrc, edst, zeros32)
        p, u = _tc_update(i < 2, sp, su, cnt, u, p,
                          wlb, blb, wrb, wlr, blr, wrr)

    lpad = L_PAD - L
    lfill = jnp.arange(lpad, dtype=jnp.int32) % N
    eli0 = jnp.concatenate([edge_label_index[0], lfill])
    eli1 = jnp.concatenate([edge_label_index[1], lfill])
    ue, pe = _sc_gather_pairs(u, p, eli0, eli1)
    pred = _tc_dot(ue, pe)
    return pred[:L, 0]


# speedup vs baseline: 8.1319x; 1.0344x over previous
import functools

import jax
import jax.numpy as jnp
from jax import lax
from jax.experimental import pallas as pl
from jax.experimental.pallas import tpu as pltpu
from jax.experimental.pallas import tpu_sc as plsc

N = 50000
D = 128
E = 625000
L = 100000

N_PAD = 50176
ROWS_PER_TILE = N_PAD // 16
CHUNK = 128
CHUNKS_PER_TILE = 312
OCTS = CHUNKS_PER_TILE // 8
E_PAD = 16 * CHUNKS_PER_TILE * CHUNK
CCHUNK = 256
CCHUNKS_PER_TILE = 156
L_PAD = 100352
LCHUNK = 112
LCHUNKS = 28

_BLK = 512
_GRID = N_PAD // _BLK

_mesh = plsc.VectorSubcoreMesh(core_axis_name="c", subcore_axis_name="s")
_f32 = jnp.float32
_sc_params = pltpu.CompilerParams(use_tc_tiling_on_sc=False,
                                  needs_layout_passes=False)


def _init_body(ux, wu, bu, px, wp, bp, ou, op):
    ou[...] = jnp.dot(ux[...], wu[...], preferred_element_type=_f32) + bu[...]
    op[...] = jnp.dot(px[...], wp[...], preferred_element_type=_f32) + bp[...]


def _tc_init(ux, wu, bu, px, wp, bp):
    spec_x = pl.BlockSpec((_BLK, D), lambda i: (i, 0))
    spec_w = pl.BlockSpec((D, D), lambda i: (0, 0))
    spec_b = pl.BlockSpec((1, D), lambda i: (0, 0))
    return pl.pallas_call(
        _init_body,
        grid=(_GRID,),
        in_specs=[spec_x, spec_w, spec_b, spec_x, spec_w, spec_b],
        out_specs=[spec_x, spec_x],
        out_shape=[jax.ShapeDtypeStruct((N_PAD, D), _f32)] * 2,
    )(ux, wu, bu.reshape(1, D), px, wp, bp.reshape(1, D))


def _update_body(relu, sp, su, cnt, u, p, wlb, blb, wrb, wlr, blr, wrr,
                 op, ou):
    mean_p = sp[...]
    mean_u = su[...]
    c = cnt[...]
    rp = 1.0 / jnp.maximum(c[0, :, 0:1], 1.0)
    ru = 1.0 / jnp.maximum(c[1, :, 0:1], 1.0)
    pn = (jnp.dot(mean_p * rp, wlb[...], preferred_element_type=_f32) + blb[...]
          + jnp.dot(p[...], wrb[...], preferred_element_type=_f32))
    un = (jnp.dot(mean_u * ru, wlr[...], preferred_element_type=_f32) + blr[...]
          + jnp.dot(u[...], wrr[...], preferred_element_type=_f32))
    if relu:
        pn = jnp.maximum(pn, 0.0)
        un = jnp.maximum(un, 0.0)
    op[...] = pn
    ou[...] = un


def _tc_update(relu, sp, su, cnt, u, p, wlb, blb, wrb, wlr, blr, wrr):
    spec_c = pl.BlockSpec((2, _BLK, 16), lambda i: (0, i, 0))
    spec_x = pl.BlockSpec((_BLK, D), lambda i: (i, 0))
    spec_w = pl.BlockSpec((D, D), lambda i: (0, 0))
    spec_b = pl.BlockSpec((1, D), lambda i: (0, 0))
    return pl.pallas_call(
        functools.partial(_update_body, relu),
        grid=(_GRID,),
        in_specs=[spec_x, spec_x, spec_c, spec_x, spec_x,
                  spec_w, spec_b, spec_w, spec_w, spec_b, spec_w],
        out_specs=[spec_x, spec_x],
        out_shape=[jax.ShapeDtypeStruct((N_PAD, D), _f32)] * 2,
    )(sp, su, cnt, u, p, wlb, blb.reshape(1, D), wrb,
      wlr, blr.reshape(1, D), wrr)


@functools.partial(
    pl.kernel,
    out_type=jax.ShapeDtypeStruct((2, N_PAD, 16), _f32),
    mesh=_mesh,
    compiler_params=_sc_params,
    scratch_types=[
        pltpu.VMEM_SHARED((N_PAD, 16), _f32),
        pltpu.VMEM((4, CCHUNK), jnp.int32),
        pltpu.VMEM((CCHUNK, 16), _f32),
        pltpu.SemaphoreType.DMA((2,)),
        pltpu.SemaphoreType.DMA((2,)),
    ],
)
def _sc_counts(esrc, edst, zeros16, out, acc, dbuf, ones, sem_i, sem_a):
    c = lax.axis_index("c")
    s = lax.axis_index("s")

    @pl.loop(0, CCHUNK)
    def _(i):
        ones[i, :] = jnp.ones((16,), _f32)

    pltpu.sync_copy(zeros16.at[pl.ds(s * ROWS_PER_TILE, ROWS_PER_TILE)],
                    acc.at[pl.ds(s * ROWS_PER_TILE, ROWS_PER_TILE)])
    plsc.subcore_barrier()
    NCH = CCHUNKS_PER_TILE
    tile_base = s * NCH * CCHUNK

    def idx_start(g, s4, s2):
        b = tile_base + g * CCHUNK

        @pl.when(c == 0)
        def _():
            pltpu.async_copy(edst.at[pl.ds(b, CCHUNK)], dbuf.at[s4],
                             sem_i.at[s2])

        @pl.when(c == 1)
        def _():
            pltpu.async_copy(esrc.at[pl.ds(b, CCHUNK)], dbuf.at[s4],
                             sem_i.at[s2])

    def idx_wait(s4, s2):
        pltpu.make_async_copy(esrc.at[pl.ds(0, CCHUNK)], dbuf.at[s4],
                              sem_i.at[s2]).wait()

    def add_wait(s4, s2):
        pltpu.make_async_copy(ones, acc.at[dbuf.at[s4]], sem_a.at[s2]).wait()

    idx_start(0, 0, 0)
    idx_start(1, 1, 1)

    @pl.loop(0, NCH // 4)
    def _(h):
        for j in range(4):
            g = 4 * h + j
            idx_wait(j, j % 2)

            @pl.when(g >= 2)
            def _():
                add_wait((j + 2) % 4, j % 2)

            @pl.when(g + 2 < NCH)
            def _():
                idx_start(g + 2, (j + 2) % 4, j % 2)

            pltpu.async_copy(ones, acc.at[dbuf.at[j]], sem_a.at[j % 2],
                             add=True)

    add_wait(2, 0)
    add_wait(3, 1)
    plsc.subcore_barrier()
    pltpu.sync_copy(acc.at[pl.ds(s * ROWS_PER_TILE, ROWS_PER_TILE)],
                    out.at[c, pl.ds(s * ROWS_PER_TILE, ROWS_PER_TILE)])


@functools.partial(
    pl.kernel,
    out_type=[jax.ShapeDtypeStruct((N_PAD, D), _f32)] * 2,
    mesh=_mesh,
    compiler_params=_sc_params,
    scratch_types=[
        pltpu.VMEM_SHARED((N_PAD, 32), _f32),
        pltpu.VMEM((4, CHUNK), jnp.int32),
        pltpu.VMEM((4, CHUNK), jnp.int32),
        pltpu.VMEM((8, CHUNK), jnp.int32),
        pltpu.VMEM((4, CHUNK, 32), _f32),
        pltpu.SemaphoreType.DMA((4,)),
        pltpu.SemaphoreType.DMA((4,)),
        pltpu.SemaphoreType.DMA((4,)),
    ],
)
def _sc_segsum(u4, p4, esrc, edst, zeros32, sp_out, su_out,
               acc, sbuf, gbuf, dbuf, rbuf, sem_i, sem_g, sem_a):
    c = lax.axis_index("c")
    s = lax.axis_index("s")
    NCH = CHUNKS_PER_TILE

    for d in range(2):
        tab = u4 if d == 0 else p4
        src_h = esrc if d == 0 else edst
        dst_h = edst if d == 0 else esrc
        out_h = sp_out if d == 0 else su_out
        for r in range(2):
            q = r * 2 + c
            pltpu.sync_copy(zeros32.at[pl.ds(s * ROWS_PER_TILE, ROWS_PER_TILE)],
                            acc.at[pl.ds(s * ROWS_PER_TILE, ROWS_PER_TILE)])
            plsc.subcore_barrier()
            tile_base = s * CHUNKS_PER_TILE * CHUNK

            def idx_start(g, s4, s8):
                b = tile_base + g * CHUNK
                pltpu.async_copy(src_h.at[pl.ds(b, CHUNK)], sbuf.at[s4],
                                 sem_i.at[s4])
                pltpu.async_copy(dst_h.at[pl.ds(b, CHUNK)], dbuf.at[s8],
                                 sem_i.at[s4])

            def idx_wait(s4, s8):
                pltpu.make_async_copy(src_h.at[pl.ds(0, CHUNK)],
                                      sbuf.at[s4], sem_i.at[s4]).wait()
                pltpu.make_async_copy(dst_h.at[pl.ds(0, CHUNK)],
                                      dbuf.at[s8], sem_i.at[s4]).wait()

            def transform(s4):
                @pl.loop(0, CHUNK, step=16)
                def _(i):
                    gbuf[s4, pl.ds(i, 16)] = sbuf[s4, pl.ds(i, 16)] * 4 + q

            def gather_start(s4):
                pltpu.async_copy(tab.at[gbuf.at[s4]], rbuf.at[s4],
                                 sem_g.at[s4])

            def gather_wait(s4):
                pltpu.make_async_copy(tab.at[gbuf.at[s4]], rbuf.at[s4],
                                      sem_g.at[s4]).wait()

            def add_start(s4, s8):
                pltpu.async_copy(rbuf.at[s4], acc.at[dbuf.at[s8]],
                                 sem_a.at[s4], add=True)

            def add_wait(s4, s8):
                pltpu.make_async_copy(rbuf.at[s4], acc.at[dbuf.at[s8]],
                                      sem_a.at[s4]).wait()

            idx_start(0, 0, 0)
            idx_start(1, 1, 1)
            idx_start(2, 2, 2)
            idx_wait(0, 0)
            transform(0)
            idx_wait(1, 1)
            transform(1)
            gather_start(0)

            @pl.loop(0, OCTS)
            def _(h):
                for j in range(8):
                    g = 8 * h + j
                    s4 = j % 4
                    if j + 1 < 8:
                        gather_start((j + 1) % 4)
                    else:
                        @pl.when(g + 1 < NCH)
                        def _():
                            gather_start((j + 1) % 4)
                    @pl.when(g + 3 < NCH)
                    def _():
                        idx_start(g + 3, (j + 3) % 4, (j + 3) % 8)
                    @pl.when(g + 2 < NCH)
                    def _():
                        idx_wait((j + 2) % 4, (j + 2) % 8)
                        transform((j + 2) % 4)
                    gather_wait(s4)
                    add_start(s4, j)
                    @pl.when(g >= 2)
                    def _():
                        add_wait((j + 2) % 4, (j + 6) % 8)

            add_wait(2, 6)
            add_wait(3, 7)
            plsc.subcore_barrier()
            pltpu.sync_copy(acc.at[pl.ds(s * ROWS_PER_TILE, ROWS_PER_TILE)],
                            out_h.at[pl.ds(s * ROWS_PER_TILE, ROWS_PER_TILE),
                                     pl.ds(q * 32, 32)])


@functools.partial(
    pl.kernel,
    out_type=[jax.ShapeDtypeStruct((L_PAD, D), _f32)] * 2,
    mesh=_mesh,
    compiler_params=_sc_params,
    scratch_types=[
        pltpu.VMEM((2, LCHUNK), jnp.int32),
        pltpu.VMEM((2, LCHUNK), jnp.int32),
        pltpu.VMEM((2, LCHUNK, D), _f32),
        pltpu.VMEM((2, LCHUNK, D), _f32),
        pltpu.SemaphoreType.DMA((2,)),
        pltpu.SemaphoreType.DMA((2,)),
        pltpu.SemaphoreType.DMA((2,)),
    ],
)
def _sc_gather_pairs(u, p, eli0, eli1, ue_out, pe_out, i0, i1, ubuf, pbuf,
                     sem_i, sem_g, sem_o):
    c = lax.axis_index("c")
    s = lax.axis_index("s")
    w = c * 16 + s
    wbase = w * ROWS_PER_TILE

    def idx_start(k, ph):
        b = wbase + k * LCHUNK
        pltpu.async_copy(eli0.at[pl.ds(b, LCHUNK)], i0.at[ph], sem_i.at[ph])
        pltpu.async_copy(eli1.at[pl.ds(b, LCHUNK)], i1.at[ph], sem_i.at[ph])

    def idx_wait(ph):
        pltpu.make_async_copy(eli0.at[pl.ds(0, LCHUNK)], i0.at[ph],
                              sem_i.at[ph]).wait()
        pltpu.make_async_copy(eli1.at[pl.ds(0, LCHUNK)], i1.at[ph],
                              sem_i.at[ph]).wait()

    def out_wait(k, ph):
        b = wbase + k * LCHUNK
        pltpu.make_async_copy(ubuf.at[ph], ue_out.at[pl.ds(b, LCHUNK)],
                              sem_o.at[ph]).wait()
        pltpu.make_async_copy(pbuf.at[ph], pe_out.at[pl.ds(b, LCHUNK)],
                              sem_o.at[ph]).wait()

    idx_start(0, 0)

    @pl.loop(0, LCHUNKS // 2)
    def _(h):
        for j in range(2):
            k = 2 * h + j
            idx_wait(j)

            @pl.when(k >= 2)
            def _():
                out_wait(k - 2, j)

            @pl.when(k + 1 < LCHUNKS)
            def _():
                idx_start(k + 1, 1 - j)

            pltpu.async_copy(u.at[i0.at[j]], ubuf.at[j], sem_g.at[j])
            pltpu.async_copy(p.at[i1.at[j]], pbuf.at[j], sem_g.at[j])
            pltpu.make_async_copy(u.at[i0.at[j]], ubuf.at[j],
                                  sem_g.at[j]).wait()
            pltpu.make_async_copy(p.at[i1.at[j]], pbuf.at[j],
                                  sem_g.at[j]).wait()
            b = wbase + k * LCHUNK
            pltpu.async_copy(ubuf.at[j], ue_out.at[pl.ds(b, LCHUNK)],
                             sem_o.at[j])
            pltpu.async_copy(pbuf.at[j], pe_out.at[pl.ds(b, LCHUNK)],
                             sem_o.at[j])

    out_wait(LCHUNKS - 2, 0)
    out_wait(LCHUNKS - 1, 1)


def _dot_body(a, b, o):
    o[...] = jnp.sum(a[...] * b[...], axis=1, keepdims=True)


def _tc_dot(ue, pe):
    spec_x = pl.BlockSpec((_BLK, D), lambda i: (i, 0))
    return pl.pallas_call(
        _dot_body,
        grid=(L_PAD // _BLK,),
        in_specs=[spec_x, spec_x],
        out_specs=pl.BlockSpec((_BLK, 1), lambda i: (i, 0)),
        out_shape=jax.ShapeDtypeStruct((L_PAD, 1), _f32),
    )(ue, pe)


def kernel(user_x, prod_x, edge_index, edge_label_index,
           W_user, b_user, W_prod, b_prod,
           W_l1_buy, b_l1_buy, W_r1_buy, W_l1_rev, b_l1_rev, W_r1_rev,
           W_l2_buy, b_l2_buy, W_r2_buy, W_l2_rev, b_l2_rev, W_r2_rev,
           W_l3_buy, b_l3_buy, W_r3_buy, W_l3_rev, b_l3_rev, W_r3_rev):
    ux = jnp.pad(user_x, ((0, N_PAD - N), (0, 0)))
    px = jnp.pad(prod_x, ((0, N_PAD - N), (0, 0)))

    npad = E_PAD - E
    fill = jnp.arange(npad, dtype=jnp.int32)
    esrc = jnp.concatenate([edge_index[0], N + fill % (N_PAD - N)])
    edst = jnp.concatenate([edge_index[1], N + (fill * 7 + 3) % (N_PAD - N)])

    zeros32 = jnp.zeros((N_PAD, 32), _f32)
    zeros16 = jnp.zeros((N_PAD, 16), _f32)

    cnt = _sc_counts(esrc, edst, zeros16)

    u, p = _tc_init(ux, W_user, b_user, px, W_prod, b_prod)

    layers = [
        (W_l1_buy, b_l1_buy, W_r1_buy, W_l1_rev, b_l1_rev, W_r1_rev),
        (W_l2_buy, b_l2_buy, W_r2_buy, W_l2_rev, b_l2_rev, W_r2_rev),
        (W_l3_buy, b_l3_buy, W_r3_buy, W_l3_rev, b_l3_rev, W_r3_rev),
    ]
    for i, (wlb, blb, wrb, wlr, blr, wrr) in enumerate(layers):
        sp, su = _sc_segsum(u.reshape(4 * N_PAD, 32), p.reshape(4 * N_PAD, 32),
                            esrc, edst, zeros32)
        p, u = _tc_update(i < 2, sp, su, cnt, u, p,
                          wlb, blb, wrb, wlr, blr, wrr)

    lpad = L_PAD - L
    lfill = jnp.arange(lpad, dtype=jnp.int32) % N
    eli0 = jnp.concatenate([edge_label_index[0], lfill])
    eli1 = jnp.concatenate([edge_label_index[1], lfill])
    ue, pe = _sc_gather_pairs(u, p, eli0, eli1)
    pred = _tc_dot(ue, pe)
    return pred[:L, 0]


# speedup vs baseline: 8.5725x; 1.0542x over previous
import functools

import jax
import jax.numpy as jnp
from jax import lax
from jax.experimental import pallas as pl
from jax.experimental.pallas import tpu as pltpu
from jax.experimental.pallas import tpu_sc as plsc

N = 50000
D = 128
E = 625000
L = 100000

N_PAD = 50176
ROWS_PER_TILE = N_PAD // 16
CHUNK = 128
CHUNKS_PER_TILE = 312
OCTS = CHUNKS_PER_TILE // 8
E_PAD = 16 * CHUNKS_PER_TILE * CHUNK
CCHUNK = 256
CCHUNKS_PER_TILE = 156
L_PAD = 100352
LCHUNK = 112
LCHUNKS = 28

_BLK = 512
_GRID = N_PAD // _BLK

_mesh = plsc.VectorSubcoreMesh(core_axis_name="c", subcore_axis_name="s")
_f32 = jnp.float32
_sc_params = pltpu.CompilerParams(use_tc_tiling_on_sc=False,
                                  needs_layout_passes=False)


def _init_body(ux, wu, bu, px, wp, bp, ou, op):
    ou[...] = jnp.dot(ux[...], wu[...], preferred_element_type=_f32) + bu[...]
    op[...] = jnp.dot(px[...], wp[...], preferred_element_type=_f32) + bp[...]


def _tc_init(ux, wu, bu, px, wp, bp):
    spec_x = pl.BlockSpec((_BLK, D), lambda i: (i, 0))
    spec_w = pl.BlockSpec((D, D), lambda i: (0, 0))
    spec_b = pl.BlockSpec((1, D), lambda i: (0, 0))
    return pl.pallas_call(
        _init_body,
        grid=(_GRID,),
        in_specs=[spec_x, spec_w, spec_b, spec_x, spec_w, spec_b],
        out_specs=[spec_x, spec_x],
        out_shape=[jax.ShapeDtypeStruct((N_PAD, D), _f32)] * 2,
    )(ux, wu, bu.reshape(1, D), px, wp, bp.reshape(1, D))


def _update_body(relu, which, sm, cnt, x, wl, bl, wr, o):
    c = cnt[...]
    recip = 1.0 / jnp.maximum(c[which, :, 0:1], 1.0)
    xn = (jnp.dot(sm[...] * recip, wl[...], preferred_element_type=_f32)
          + bl[...] + jnp.dot(x[...], wr[...], preferred_element_type=_f32))
    if relu:
        xn = jnp.maximum(xn, 0.0)
    o[...] = xn


def _tc_update_one(relu, which, sm, cnt, x, wl, bl, wr):
    spec_c = pl.BlockSpec((2, _BLK, 16), lambda i: (0, i, 0))
    spec_x = pl.BlockSpec((_BLK, D), lambda i: (i, 0))
    spec_w = pl.BlockSpec((D, D), lambda i: (0, 0))
    spec_b = pl.BlockSpec((1, D), lambda i: (0, 0))
    return pl.pallas_call(
        functools.partial(_update_body, relu, which),
        grid=(_GRID,),
        in_specs=[spec_x, spec_c, spec_x, spec_w, spec_b, spec_w],
        out_specs=spec_x,
        out_shape=jax.ShapeDtypeStruct((N_PAD, D), _f32),
    )(sm, cnt, x, wl, bl.reshape(1, D), wr)


@functools.partial(
    pl.kernel,
    out_type=jax.ShapeDtypeStruct((2, N_PAD, 16), _f32),
    mesh=_mesh,
    compiler_params=_sc_params,
    scratch_types=[
        pltpu.VMEM_SHARED((N_PAD, 16), _f32),
        pltpu.VMEM((4, CCHUNK), jnp.int32),
        pltpu.VMEM((CCHUNK, 16), _f32),
        pltpu.SemaphoreType.DMA((2,)),
        pltpu.SemaphoreType.DMA((2,)),
    ],
)
def _sc_counts(esrc, edst, zeros16, out, acc, dbuf, ones, sem_i, sem_a):
    c = lax.axis_index("c")
    s = lax.axis_index("s")

    @pl.loop(0, CCHUNK)
    def _(i):
        ones[i, :] = jnp.ones((16,), _f32)

    pltpu.sync_copy(zeros16.at[pl.ds(s * ROWS_PER_TILE, ROWS_PER_TILE)],
                    acc.at[pl.ds(s * ROWS_PER_TILE, ROWS_PER_TILE)])
    plsc.subcore_barrier()
    NCH = CCHUNKS_PER_TILE
    tile_base = s * NCH * CCHUNK

    def idx_start(g, s4, s2):
        b = tile_base + g * CCHUNK

        @pl.when(c == 0)
        def _():
            pltpu.async_copy(edst.at[pl.ds(b, CCHUNK)], dbuf.at[s4],
                             sem_i.at[s2])

        @pl.when(c == 1)
        def _():
            pltpu.async_copy(esrc.at[pl.ds(b, CCHUNK)], dbuf.at[s4],
                             sem_i.at[s2])

    def idx_wait(s4, s2):
        pltpu.make_async_copy(esrc.at[pl.ds(0, CCHUNK)], dbuf.at[s4],
                              sem_i.at[s2]).wait()

    def add_wait(s4, s2):
        pltpu.make_async_copy(ones, acc.at[dbuf.at[s4]], sem_a.at[s2]).wait()

    idx_start(0, 0, 0)
    idx_start(1, 1, 1)

    @pl.loop(0, NCH // 4)
    def _(h):
        for j in range(4):
            g = 4 * h + j
            idx_wait(j, j % 2)

            @pl.when(g >= 2)
            def _():
                add_wait((j + 2) % 4, j % 2)

            @pl.when(g + 2 < NCH)
            def _():
                idx_start(g + 2, (j + 2) % 4, j % 2)

            pltpu.async_copy(ones, acc.at[dbuf.at[j]], sem_a.at[j % 2],
                             add=True)

    add_wait(2, 0)
    add_wait(3, 1)
    plsc.subcore_barrier()
    pltpu.sync_copy(acc.at[pl.ds(s * ROWS_PER_TILE, ROWS_PER_TILE)],
                    out.at[c, pl.ds(s * ROWS_PER_TILE, ROWS_PER_TILE)])


@functools.partial(
    pl.kernel,
    out_type=jax.ShapeDtypeStruct((N_PAD, D), _f32),
    mesh=_mesh,
    compiler_params=_sc_params,
    scratch_types=[
        pltpu.VMEM_SHARED((N_PAD, 32), _f32),
        pltpu.VMEM((4, CHUNK), jnp.int32),
        pltpu.VMEM((4, CHUNK), jnp.int32),
        pltpu.VMEM((8, CHUNK), jnp.int32),
        pltpu.VMEM((4, CHUNK, 32), _f32),
        pltpu.SemaphoreType.DMA((4,)),
        pltpu.SemaphoreType.DMA((4,)),
        pltpu.SemaphoreType.DMA((4,)),
    ],
)
def _sc_segsum_one(tab, src_h, dst_h, zeros32, out_h,
                   acc, sbuf, gbuf, dbuf, rbuf, sem_i, sem_g, sem_a):
    c = lax.axis_index("c")
    s = lax.axis_index("s")
    NCH = CHUNKS_PER_TILE

    if True:
        for r in range(2):
            q = r * 2 + c
            pltpu.sync_copy(zeros32.at[pl.ds(s * ROWS_PER_TILE, ROWS_PER_TILE)],
                            acc.at[pl.ds(s * ROWS_PER_TILE, ROWS_PER_TILE)])
            plsc.subcore_barrier()
            tile_base = s * CHUNKS_PER_TILE * CHUNK

            def idx_start(g, s4, s8):
                b = tile_base + g * CHUNK
                pltpu.async_copy(src_h.at[pl.ds(b, CHUNK)], sbuf.at[s4],
                                 sem_i.at[s4])
                pltpu.async_copy(dst_h.at[pl.ds(b, CHUNK)], dbuf.at[s8],
                                 sem_i.at[s4])

            def idx_wait(s4, s8):
                pltpu.make_async_copy(src_h.at[pl.ds(0, CHUNK)],
                                      sbuf.at[s4], sem_i.at[s4]).wait()
                pltpu.make_async_copy(dst_h.at[pl.ds(0, CHUNK)],
                                      dbuf.at[s8], sem_i.at[s4]).wait()

            def transform(s4):
                @pl.loop(0, CHUNK, step=16)
                def _(i):
                    gbuf[s4, pl.ds(i, 16)] = sbuf[s4, pl.ds(i, 16)] * 4 + q

            def gather_start(s4):
                pltpu.async_copy(tab.at[gbuf.at[s4]], rbuf.at[s4],
                                 sem_g.at[s4])

            def gather_wait(s4):
                pltpu.make_async_copy(tab.at[gbuf.at[s4]], rbuf.at[s4],
                                      sem_g.at[s4]).wait()

            def add_start(s4, s8):
                pltpu.async_copy(rbuf.at[s4], acc.at[dbuf.at[s8]],
                                 sem_a.at[s4], add=True)

            def add_wait(s4, s8):
                pltpu.make_async_copy(rbuf.at[s4], acc.at[dbuf.at[s8]],
                                      sem_a.at[s4]).wait()

            idx_start(0, 0, 0)
            idx_start(1, 1, 1)
            idx_start(2, 2, 2)
            idx_wait(0, 0)
            transform(0)
            idx_wait(1, 1)
            transform(1)
            gather_start(0)

            @pl.loop(0, OCTS)
            def _(h):
                for j in range(8):
                    g = 8 * h + j
                    s4 = j % 4
                    if j + 1 < 8:
                        gather_start((j + 1) % 4)
                    else:
                        @pl.when(g + 1 < NCH)
                        def _():
                            gather_start((j + 1) % 4)
                    @pl.when(g + 3 < NCH)
                    def _():
                        idx_start(g + 3, (j + 3) % 4, (j + 3) % 8)
                    @pl.when(g + 2 < NCH)
                    def _():
                        idx_wait((j + 2) % 4, (j + 2) % 8)
                        transform((j + 2) % 4)
                    gather_wait(s4)
                    add_start(s4, j)
                    @pl.when(g >= 2)
                    def _():
                        add_wait((j + 2) % 4, (j + 6) % 8)

            add_wait(2, 6)
            add_wait(3, 7)
            plsc.subcore_barrier()
            pltpu.sync_copy(acc.at[pl.ds(s * ROWS_PER_TILE, ROWS_PER_TILE)],
                            out_h.at[pl.ds(s * ROWS_PER_TILE, ROWS_PER_TILE),
                                     pl.ds(q * 32, 32)])


@functools.partial(
    pl.kernel,
    out_type=[jax.ShapeDtypeStruct((L_PAD, D), _f32)] * 2,
    mesh=_mesh,
    compiler_params=_sc_params,
    scratch_types=[
        pltpu.VMEM((2, LCHUNK), jnp.int32),
        pltpu.VMEM((2, LCHUNK), jnp.int32),
        pltpu.VMEM((2, LCHUNK, D), _f32),
        pltpu.VMEM((2, LCHUNK, D), _f32),
        pltpu.SemaphoreType.DMA((2,)),
        pltpu.SemaphoreType.DMA((2,)),
        pltpu.SemaphoreType.DMA((2,)),
    ],
)
def _sc_gather_pairs(u, p, eli0, eli1, ue_out, pe_out, i0, i1, ubuf, pbuf,
                     sem_i, sem_g, sem_o):
    c = lax.axis_index("c")
    s = lax.axis_index("s")
    w = c * 16 + s
    wbase = w * ROWS_PER_TILE

    def idx_start(k, ph):
        b = wbase + k * LCHUNK
        pltpu.async_copy(eli0.at[pl.ds(b, LCHUNK)], i0.at[ph], sem_i.at[ph])
        pltpu.async_copy(eli1.at[pl.ds(b, LCHUNK)], i1.at[ph], sem_i.at[ph])

    def idx_wait(ph):
        pltpu.make_async_copy(eli0.at[pl.ds(0, LCHUNK)], i0.at[ph],
                              sem_i.at[ph]).wait()
        pltpu.make_async_copy(eli1.at[pl.ds(0, LCHUNK)], i1.at[ph],
                              sem_i.at[ph]).wait()

    def out_wait(k, ph):
        b = wbase + k * LCHUNK
        pltpu.make_async_copy(ubuf.at[ph], ue_out.at[pl.ds(b, LCHUNK)],
                              sem_o.at[ph]).wait()
        pltpu.make_async_copy(pbuf.at[ph], pe_out.at[pl.ds(b, LCHUNK)],
                              sem_o.at[ph]).wait()

    idx_start(0, 0)

    @pl.loop(0, LCHUNKS // 2)
    def _(h):
        for j in range(2):
            k = 2 * h + j
            idx_wait(j)

            @pl.when(k >= 2)
            def _():
                out_wait(k - 2, j)

            @pl.when(k + 1 < LCHUNKS)
            def _():
                idx_start(k + 1, 1 - j)

            pltpu.async_copy(u.at[i0.at[j]], ubuf.at[j], sem_g.at[j])
            pltpu.async_copy(p.at[i1.at[j]], pbuf.at[j], sem_g.at[j])
            pltpu.make_async_copy(u.at[i0.at[j]], ubuf.at[j],
                                  sem_g.at[j]).wait()
            pltpu.make_async_copy(p.at[i1.at[j]], pbuf.at[j],
                                  sem_g.at[j]).wait()
            b = wbase + k * LCHUNK
            pltpu.async_copy(ubuf.at[j], ue_out.at[pl.ds(b, LCHUNK)],
                             sem_o.at[j])
            pltpu.async_copy(pbuf.at[j], pe_out.at[pl.ds(b, LCHUNK)],
                             sem_o.at[j])

    out_wait(LCHUNKS - 2, 0)
    out_wait(LCHUNKS - 1, 1)


def _dot_body(a, b, o):
    o[...] = jnp.sum(a[...] * b[...], axis=1, keepdims=True)


def _tc_dot(ue, pe):
    spec_x = pl.BlockSpec((_BLK, D), lambda i: (i, 0))
    return pl.pallas_call(
        _dot_body,
        grid=(L_PAD // _BLK,),
        in_specs=[spec_x, spec_x],
        out_specs=pl.BlockSpec((_BLK, 1), lambda i: (i, 0)),
        out_shape=jax.ShapeDtypeStruct((L_PAD, 1), _f32),
    )(ue, pe)


def kernel(user_x, prod_x, edge_index, edge_label_index,
           W_user, b_user, W_prod, b_prod,
           W_l1_buy, b_l1_buy, W_r1_buy, W_l1_rev, b_l1_rev, W_r1_rev,
           W_l2_buy, b_l2_buy, W_r2_buy, W_l2_rev, b_l2_rev, W_r2_rev,
           W_l3_buy, b_l3_buy, W_r3_buy, W_l3_rev, b_l3_rev, W_r3_rev):
    ux = jnp.pad(user_x, ((0, N_PAD - N), (0, 0)))
    px = jnp.pad(prod_x, ((0, N_PAD - N), (0, 0)))

    npad = E_PAD - E
    fill = jnp.arange(npad, dtype=jnp.int32)
    esrc = jnp.concatenate([edge_index[0], N + fill % (N_PAD - N)])
    edst = jnp.concatenate([edge_index[1], N + (fill * 7 + 3) % (N_PAD - N)])

    zeros32 = jnp.zeros((N_PAD, 32), _f32)
    zeros16 = jnp.zeros((N_PAD, 16), _f32)

    cnt = _sc_counts(esrc, edst, zeros16)

    u, p = _tc_init(ux, W_user, b_user, px, W_prod, b_prod)

    layers = [
        (W_l1_buy, b_l1_buy, W_r1_buy, W_l1_rev, b_l1_rev, W_r1_rev),
        (W_l2_buy, b_l2_buy, W_r2_buy, W_l2_rev, b_l2_rev, W_r2_rev),
        (W_l3_buy, b_l3_buy, W_r3_buy, W_l3_rev, b_l3_rev, W_r3_rev),
    ]
    for i, (wlb, blb, wrb, wlr, blr, wrr) in enumerate(layers):
        sp = _sc_segsum_one(u.reshape(4 * N_PAD, 32), esrc, edst, zeros32)
        su = _sc_segsum_one(p.reshape(4 * N_PAD, 32), edst, esrc, zeros32)
        p_new = _tc_update_one(i < 2, 0, sp, cnt, p, wlb, blb, wrb)
        u_new = _tc_update_one(i < 2, 1, su, cnt, u, wlr, blr, wrr)
        p, u = p_new, u_new

    lpad = L_PAD - L
    lfill = jnp.arange(lpad, dtype=jnp.int32) % N
    eli0 = jnp.concatenate([edge_label_index[0], lfill])
    eli1 = jnp.concatenate([edge_label_index[1], lfill])
    ue, pe = _sc_gather_pairs(u, p, eli0, eli1)
    pred = _tc_dot(ue, pe)
    return pred[:L, 0]
